# Initial kernel scaffold; baseline (speedup 1.0000x reference)
import functools

import jax
import jax.numpy as jnp
from jax import lax
from jax.experimental import pallas as pl
from jax.experimental.pallas import tpu as pltpu
from jax.experimental.pallas import tpu_sc as plsc

N = 10000
DIN = 128
DOUT = 128
E = 320000

NC = 2
NS = 16
NW = NC * NS
PER_W = E // NW
CHUNK = 80
NCHUNKS = PER_W // CHUNK
ROWS_PER_TILE = N // NS
ZROWS = 125


def _mm_body(x_ref, w_ref, o_ref):
    o_ref[...] = jnp.dot(x_ref[...], w_ref[...],
                         preferred_element_type=jnp.float32)


def _sum_body(p0_ref, p1_ref, b_ref, o_ref):
    o_ref[...] = p0_ref[...] + p1_ref[...] + b_ref[...]


def _edge_body(xw_hbm, src_hbm, dst_hbm, out_hbm,
               src_v, dst_v, rows_v, zbuf, acc, gsem):
    cid = lax.axis_index("c")
    sid = lax.axis_index("s")
    wid = sid * NC + cid

    def zrow(r, carry):
        for j in range(DOUT // 16):
            zbuf[r, pl.ds(j * 16, 16)] = jnp.zeros((16,), jnp.float32)
        return carry
    lax.fori_loop(0, ZROWS, zrow, 0)
    for k in range(ROWS_PER_TILE // ZROWS):
        pltpu.sync_copy(zbuf, acc.at[pl.ds(sid * ROWS_PER_TILE + k * ZROWS,
                                           ZROWS)])
    plsc.subcore_barrier()

    def step(i, carry):
        base = wid * PER_W + i * CHUNK
        pltpu.sync_copy(src_hbm.at[pl.ds(base, CHUNK)], src_v)
        pltpu.sync_copy(dst_hbm.at[pl.ds(base, CHUNK)], dst_v)
        pltpu.async_copy(xw_hbm.at[src_v], rows_v, gsem).wait()
        pltpu.sync_copy(rows_v, acc.at[dst_v], add=True)
        return carry
    lax.fori_loop(0, NCHUNKS, step, 0)
    plsc.subcore_barrier()

    r0 = sid * ROWS_PER_TILE
    pltpu.sync_copy(acc.at[pl.ds(r0, ROWS_PER_TILE)],
                    out_hbm.at[pl.ds(cid * N + r0, ROWS_PER_TILE)])


@jax.jit
def _gcn(x, edge_index, W, b):
    xw = pl.pallas_call(
        _mm_body,
        grid=(8,),
        in_specs=[pl.BlockSpec((N // 8, DIN), lambda i: (i, 0)),
                  pl.BlockSpec((DIN, DOUT), lambda i: (0, 0))],
        out_specs=pl.BlockSpec((N // 8, DOUT), lambda i: (i, 0)),
        out_shape=jax.ShapeDtypeStruct((N, DOUT), jnp.float32),
    )(x, W)

    src = edge_index[0]
    dst = edge_index[1]

    edge_kernel = pl.kernel(
        _edge_body,
        out_type=jax.ShapeDtypeStruct((2 * N, DOUT), jnp.float32),
        mesh=plsc.VectorSubcoreMesh(core_axis_name="c", subcore_axis_name="s"),
        scratch_types=[
            pltpu.VMEM((CHUNK,), jnp.int32),
            pltpu.VMEM((CHUNK,), jnp.int32),
            pltpu.VMEM((CHUNK, DOUT), jnp.float32),
            pltpu.VMEM((ZROWS, DOUT), jnp.float32),
            pltpu.VMEM_SHARED((N, DOUT), jnp.float32),
            pltpu.SemaphoreType.DMA,
        ],
    )
    partials = edge_kernel(xw, src, dst)

    out = pl.pallas_call(
        _sum_body,
        grid=(8,),
        in_specs=[pl.BlockSpec((N // 8, DOUT), lambda i: (i, 0)),
                  pl.BlockSpec((N // 8, DOUT), lambda i: (i + 8, 0)),
                  pl.BlockSpec((1, DOUT), lambda i: (0, 0))],
        out_specs=pl.BlockSpec((N // 8, DOUT), lambda i: (i, 0)),
        out_shape=jax.ShapeDtypeStruct((N, DOUT), jnp.float32),
    )(partials, partials, b.reshape(1, DOUT))
    return out


def kernel(x, edge_index, edge_attr, return_attention_weights, W, b):
    out = _gcn(x, edge_index, W, b)
    return (out, (None, None))


# speedup vs baseline: 5.4111x; 5.4111x over previous
import functools

import jax
import jax.numpy as jnp
from jax import lax
from jax.experimental import pallas as pl
from jax.experimental.pallas import tpu as pltpu
from jax.experimental.pallas import tpu_sc as plsc

N = 10000
DIN = 128
DOUT = 128
E = 320000

NC = 2
NS = 16
NW = NC * NS
PER_W = E // NW
CHUNK = 80
NCHUNKS = PER_W // CHUNK
WB_TILES = 10
ROWS_PER_WB = N // WB_TILES
ZROWS = 200


def _mm_body(x_ref, w_ref, o_ref):
    o_ref[...] = jnp.dot(x_ref[...], w_ref[...],
                         preferred_element_type=jnp.float32)


def _sum_body(p0_ref, p1_ref, b_ref, o_ref):
    o_ref[...] = p0_ref[...] + p1_ref[...] + b_ref[...]


def _edge_body(xw_hbm, src_hbm, dst_hbm, out_hbm,
               src_v, dst_v, rows_v, zbuf, acc, gsem):
    cid = lax.axis_index("c")
    sid = lax.axis_index("s")
    wid = sid * NC + cid

    @pl.when(sid < WB_TILES)
    def _zero():
        def zrow(r, carry):
            for j in range(DOUT // 16):
                zbuf[r, pl.ds(j * 16, 16)] = jnp.zeros((16,), jnp.float32)
            return carry
        lax.fori_loop(0, ZROWS, zrow, 0)
        for k in range(ROWS_PER_WB // ZROWS):
            pltpu.sync_copy(zbuf, acc.at[pl.ds(sid * ROWS_PER_WB + k * ZROWS,
                                               ZROWS)])
    plsc.subcore_barrier()

    def step(i, carry):
        base = wid * PER_W + i * CHUNK
        pltpu.sync_copy(src_hbm.at[pl.ds(base, CHUNK)], src_v)
        pltpu.sync_copy(dst_hbm.at[pl.ds(base, CHUNK)], dst_v)
        pltpu.async_copy(xw_hbm.at[src_v], rows_v, gsem).wait()
        pltpu.sync_copy(rows_v, acc.at[dst_v], add=True)
        return carry
    lax.fori_loop(0, NCHUNKS, step, 0)
    plsc.subcore_barrier()

    @pl.when(sid < WB_TILES)
    def _writeback():
        r0 = sid * ROWS_PER_WB
        pltpu.sync_copy(acc.at[pl.ds(r0, ROWS_PER_WB)],
                        out_hbm.at[pl.ds(cid * N + r0, ROWS_PER_WB)])


@jax.jit
def _gcn(x, edge_index, W, b):
    xw = pl.pallas_call(
        _mm_body,
        grid=(10,),
        in_specs=[pl.BlockSpec((N // 10, DIN), lambda i: (i, 0)),
                  pl.BlockSpec((DIN, DOUT), lambda i: (0, 0))],
        out_specs=pl.BlockSpec((N // 10, DOUT), lambda i: (i, 0)),
        out_shape=jax.ShapeDtypeStruct((N, DOUT), jnp.float32),
    )(x, W)

    src = edge_index[0]
    dst = edge_index[1]

    edge_kernel = pl.kernel(
        _edge_body,
        out_type=jax.ShapeDtypeStruct((2 * N, DOUT), jnp.float32),
        mesh=plsc.VectorSubcoreMesh(core_axis_name="c", subcore_axis_name="s"),
        scratch_types=[
            pltpu.VMEM((CHUNK,), jnp.int32),
            pltpu.VMEM((CHUNK,), jnp.int32),
            pltpu.VMEM((CHUNK, DOUT), jnp.float32),
            pltpu.VMEM((ZROWS, DOUT), jnp.float32),
            pltpu.VMEM_SHARED((N, DOUT), jnp.float32),
            pltpu.SemaphoreType.DMA,
        ],
    )
    partials = edge_kernel(xw, src, dst)

    out = pl.pallas_call(
        _sum_body,
        grid=(10,),
        in_specs=[pl.BlockSpec((N // 10, DOUT), lambda i: (i, 0)),
                  pl.BlockSpec((N // 10, DOUT), lambda i: (i + 10, 0)),
                  pl.BlockSpec((1, DOUT), lambda i: (0, 0))],
        out_specs=pl.BlockSpec((N // 10, DOUT), lambda i: (i, 0)),
        out_shape=jax.ShapeDtypeStruct((N, DOUT), jnp.float32),
    )(partials, partials, b.reshape(1, DOUT))
    return out


def kernel(x, edge_index, edge_attr, return_attention_weights, W, b):
    out = _gcn(x, edge_index, W, b)
    return (out, (None, None))


# speedup vs baseline: 11.4386x; 2.1139x over previous
import functools

import jax
import jax.numpy as jnp
from jax import lax
from jax.experimental import pallas as pl
from jax.experimental.pallas import tpu as pltpu
from jax.experimental.pallas import tpu_sc as plsc

N = 10000
DIN = 128
DOUT = 128
E = 320000

NC = 2
NS = 16
NW = NC * NS
PER_W = E // NW
CHUNK = 80
NCHUNKS = PER_W // CHUNK
WB_TILES = 10
ROWS_PER_WB = N // WB_TILES
ZROWS = 40


def _mm_body(x_ref, w_ref, o_ref):
    o_ref[...] = jnp.dot(x_ref[...], w_ref[...],
                         preferred_element_type=jnp.float32)


def _sum_body(p0_ref, p1_ref, b_ref, o_ref):
    o_ref[...] = p0_ref[...] + p1_ref[...] + b_ref[...]


def _edge_body(xw_hbm, src_hbm, dst_hbm, out_hbm,
               src_v, dst_a, dst_b, rows_a, rows_b, acc,
               gsem_a, gsem_b, dsem_a, dsem_b, ssem_a, ssem_b):
    cid = lax.axis_index("c")
    sid = lax.axis_index("s")
    wid = sid * NC + cid

    @pl.when(sid < WB_TILES)
    def _zero():
        def zrow(r, carry):
            for j in range(DOUT // 16):
                rows_a[r, pl.ds(j * 16, 16)] = jnp.zeros((16,), jnp.float32)
            return carry
        lax.fori_loop(0, ZROWS, zrow, 0)
        for k in range(ROWS_PER_WB // ZROWS):
            pltpu.sync_copy(rows_a.at[pl.ds(0, ZROWS)],
                            acc.at[pl.ds(sid * ROWS_PER_WB + k * ZROWS,
                                         ZROWS)])
    plsc.subcore_barrier()

    pltpu.sync_copy(src_hbm.at[wid], src_v)

    def issue(c, rows, dstv, gsem, dsem):
        pltpu.async_copy(xw_hbm.at[src_v.at[c]], rows, gsem)
        pltpu.async_copy(dst_hbm.at[pl.ds(wid * PER_W + c * CHUNK, CHUNK)],
                         dstv, dsem)

    def drain(rows, dstv, gsem, dsem):
        pltpu.make_async_copy(xw_hbm.at[pl.ds(0, CHUNK)], rows, gsem).wait()
        pltpu.make_async_copy(dst_hbm.at[pl.ds(0, CHUNK)], dstv, dsem).wait()

    def scatter(rows, dstv, ssem):
        pltpu.async_copy(rows, acc.at[dstv], ssem, add=True).wait()

    issue(0, rows_a, dst_a, gsem_a, dsem_a)

    def pair(o, carry):
        issue(2 * o + 1, rows_b, dst_b, gsem_b, dsem_b)
        drain(rows_a, dst_a, gsem_a, dsem_a)
        scatter(rows_a, dst_a, ssem_a)
        issue(2 * o + 2, rows_a, dst_a, gsem_a, dsem_a)
        drain(rows_b, dst_b, gsem_b, dsem_b)
        scatter(rows_b, dst_b, ssem_b)
        return carry
    lax.fori_loop(0, NCHUNKS // 2, pair, 0)

    drain(rows_a, dst_a, gsem_a, dsem_a)
    scatter(rows_a, dst_a, ssem_a)
    plsc.subcore_barrier()

    @pl.when(sid < WB_TILES)
    def _writeback():
        r0 = sid * ROWS_PER_WB
        pltpu.sync_copy(acc.at[pl.ds(r0, ROWS_PER_WB)],
                        out_hbm.at[pl.ds(cid * N + r0, ROWS_PER_WB)])


@jax.jit
def _gcn(x, edge_index, W, b):
    xw = pl.pallas_call(
        _mm_body,
        grid=(10,),
        in_specs=[pl.BlockSpec((N // 10, DIN), lambda i: (i, 0)),
                  pl.BlockSpec((DIN, DOUT), lambda i: (0, 0))],
        out_specs=pl.BlockSpec((N // 10, DOUT), lambda i: (i, 0)),
        out_shape=jax.ShapeDtypeStruct((N, DOUT), jnp.float32),
    )(x, W)

    src = edge_index[0].reshape(NW, NCHUNKS, CHUNK)
    dst = edge_index[1]

    edge_kernel = pl.kernel(
        _edge_body,
        out_type=jax.ShapeDtypeStruct((2 * N, DOUT), jnp.float32),
        mesh=plsc.VectorSubcoreMesh(core_axis_name="c", subcore_axis_name="s"),
        scratch_types=[
            pltpu.VMEM((NCHUNKS, CHUNK), jnp.int32),
            pltpu.VMEM((CHUNK,), jnp.int32),
            pltpu.VMEM((CHUNK,), jnp.int32),
            pltpu.VMEM((CHUNK, DOUT), jnp.float32),
            pltpu.VMEM((CHUNK, DOUT), jnp.float32),
            pltpu.VMEM_SHARED((N, DOUT), jnp.float32),
            pltpu.SemaphoreType.DMA,
            pltpu.SemaphoreType.DMA,
            pltpu.SemaphoreType.DMA,
            pltpu.SemaphoreType.DMA,
            pltpu.SemaphoreType.DMA,
            pltpu.SemaphoreType.DMA,
        ],
    )
    partials = edge_kernel(xw, src, dst)

    out = pl.pallas_call(
        _sum_body,
        grid=(10,),
        in_specs=[pl.BlockSpec((N // 10, DOUT), lambda i: (i, 0)),
                  pl.BlockSpec((N // 10, DOUT), lambda i: (i + 10, 0)),
                  pl.BlockSpec((1, DOUT), lambda i: (0, 0))],
        out_specs=pl.BlockSpec((N // 10, DOUT), lambda i: (i, 0)),
        out_shape=jax.ShapeDtypeStruct((N, DOUT), jnp.float32),
    )(partials, partials, b.reshape(1, DOUT))
    return out


def kernel(x, edge_index, edge_attr, return_attention_weights, W, b):
    out = _gcn(x, edge_index, W, b)
    return (out, (None, None))


# speedup vs baseline: 12.8232x; 1.1211x over previous
import functools

import jax
import jax.numpy as jnp
from jax import lax
from jax.experimental import pallas as pl
from jax.experimental.pallas import tpu as pltpu
from jax.experimental.pallas import tpu_sc as plsc

N = 10000
DIN = 128
DOUT = 128
E = 320000

NC = 2
NS = 16
NW = NC * NS
PER_W = E // NW
CHUNK = 80
NCHUNKS = PER_W // CHUNK
WB_TILES = 10
ROWS_PER_WB = N // WB_TILES
ZROWS = 40


def _mm_body(x_ref, w_ref, o_ref):
    o_ref[...] = jnp.dot(x_ref[...], w_ref[...],
                         preferred_element_type=jnp.float32)


def _sum_body(p0_ref, p1_ref, b_ref, o_ref):
    o_ref[...] = p0_ref[...] + p1_ref[...] + b_ref[...]


def _edge_body(xw_hbm, src_hbm, dst_hbm, out_hbm,
               src_v, dst0, dst1, dst2, rows0, rows1, rows2, acc,
               g0, g1, g2, d0, d1, d2, s0, s1, s2):
    cid = lax.axis_index("c")
    sid = lax.axis_index("s")
    wid = sid * NC + cid
    ROWS, DST = [rows0, rows1, rows2], [dst0, dst1, dst2]
    G, D, S = [g0, g1, g2], [d0, d1, d2], [s0, s1, s2]

    @pl.when(sid < WB_TILES)
    def _zero():
        def zrow(r, carry):
            for j in range(DOUT // 16):
                rows0[r, pl.ds(j * 16, 16)] = jnp.zeros((16,), jnp.float32)
            return carry
        lax.fori_loop(0, ZROWS, zrow, 0)
        for k in range(ROWS_PER_WB // ZROWS):
            pltpu.sync_copy(rows0.at[pl.ds(0, ZROWS)],
                            acc.at[pl.ds(sid * ROWS_PER_WB + k * ZROWS,
                                         ZROWS)])
    plsc.subcore_barrier()

    pltpu.sync_copy(src_hbm.at[wid], src_v)

    def issue(c, b):
        pltpu.async_copy(xw_hbm.at[src_v.at[c]], ROWS[b], G[b])
        pltpu.async_copy(dst_hbm.at[pl.ds(wid * PER_W + c * CHUNK, CHUNK)],
                         DST[b], D[b])

    def wait_g(b):
        pltpu.make_async_copy(xw_hbm.at[pl.ds(0, CHUNK)], ROWS[b],
                              G[b]).wait()
        pltpu.make_async_copy(dst_hbm.at[pl.ds(0, CHUNK)], DST[b],
                              D[b]).wait()

    def issue_scatter(b):
        pltpu.async_copy(ROWS[b], acc.at[DST[b]], S[b], add=True)

    def wait_s(b):
        pltpu.make_async_copy(ROWS[b], acc.at[DST[b]], S[b]).wait()

    issue(0, 0)
    issue(1, 1)
    wait_g(0)
    issue_scatter(0)
    issue(2, 2)

    def triple(o, carry):
        c = 3 * o
        wait_g(1); issue_scatter(1); wait_s(0); issue(c + 3, 0)
        wait_g(2); issue_scatter(2); wait_s(1); issue(c + 4, 1)
        wait_g(0); issue_scatter(0); wait_s(2); issue(c + 5, 2)
        return carry
    lax.fori_loop(0, (NCHUNKS - 5) // 3, triple, 0)

    wait_g(1); issue_scatter(1); wait_s(0); issue(NCHUNKS - 2, 0)
    wait_g(2); issue_scatter(2); wait_s(1); issue(NCHUNKS - 1, 1)
    wait_g(0); issue_scatter(0); wait_s(2)
    wait_g(1); issue_scatter(1); wait_s(0)
    wait_s(1)
    plsc.subcore_barrier()

    @pl.when(sid < WB_TILES)
    def _writeback():
        r0 = sid * ROWS_PER_WB
        pltpu.sync_copy(acc.at[pl.ds(r0, ROWS_PER_WB)],
                        out_hbm.at[pl.ds(cid * N + r0, ROWS_PER_WB)])


@jax.jit
def _gcn(x, edge_index, W, b):
    xw = pl.pallas_call(
        _mm_body,
        grid=(10,),
        in_specs=[pl.BlockSpec((N // 10, DIN), lambda i: (i, 0)),
                  pl.BlockSpec((DIN, DOUT), lambda i: (0, 0))],
        out_specs=pl.BlockSpec((N // 10, DOUT), lambda i: (i, 0)),
        out_shape=jax.ShapeDtypeStruct((N, DOUT), jnp.float32),
    )(x, W)

    src = edge_index[0].reshape(NW, NCHUNKS, CHUNK)
    dst = edge_index[1]

    edge_kernel = pl.kernel(
        _edge_body,
        out_type=jax.ShapeDtypeStruct((2 * N, DOUT), jnp.float32),
        mesh=plsc.VectorSubcoreMesh(core_axis_name="c", subcore_axis_name="s"),
        scratch_types=[
            pltpu.VMEM((NCHUNKS, CHUNK), jnp.int32),
            pltpu.VMEM((CHUNK,), jnp.int32),
            pltpu.VMEM((CHUNK,), jnp.int32),
            pltpu.VMEM((CHUNK,), jnp.int32),
            pltpu.VMEM((CHUNK, DOUT), jnp.float32),
            pltpu.VMEM((CHUNK, DOUT), jnp.float32),
            pltpu.VMEM((CHUNK, DOUT), jnp.float32),
            pltpu.VMEM_SHARED((N, DOUT), jnp.float32),
        ] + [pltpu.SemaphoreType.DMA] * 9,
    )
    partials = edge_kernel(xw, src, dst)

    out = pl.pallas_call(
        _sum_body,
        grid=(10,),
        in_specs=[pl.BlockSpec((N // 10, DOUT), lambda i: (i, 0)),
                  pl.BlockSpec((N // 10, DOUT), lambda i: (i + 10, 0)),
                  pl.BlockSpec((1, DOUT), lambda i: (0, 0))],
        out_specs=pl.BlockSpec((N // 10, DOUT), lambda i: (i, 0)),
        out_shape=jax.ShapeDtypeStruct((N, DOUT), jnp.float32),
    )(partials, partials, b.reshape(1, DOUT))
    return out


def kernel(x, edge_index, edge_attr, return_attention_weights, W, b):
    out = _gcn(x, edge_index, W, b)
    return (out, (None, None))


# speedup vs baseline: 13.5768x; 1.0588x over previous
import functools

import jax
import jax.numpy as jnp
from jax import lax
from jax.experimental import pallas as pl
from jax.experimental.pallas import tpu as pltpu
from jax.experimental.pallas import tpu_sc as plsc

N = 10000
DIN = 128
DOUT = 128
E = 320000

NC = 2
NS = 16
NW = NC * NS
PER_W = E // NW
CHUNK = 80
NCHUNKS = PER_W // CHUNK
WB_TILES = 10
ROWS_PER_WB = N // WB_TILES
ZROWS = 40


def _fin_body(p0_ref, p1_ref, w_ref, b_ref, o_ref):
    o_ref[...] = jnp.dot(p0_ref[...] + p1_ref[...], w_ref[...],
                         preferred_element_type=jnp.float32) + b_ref[...]


def _edge_body(xw_hbm, src_hbm, dst_hbm, out_hbm,
               src_v, dst0, dst1, dst2, rows0, rows1, rows2, acc,
               g0, g1, g2, d0, d1, d2, s0, s1, s2):
    cid = lax.axis_index("c")
    sid = lax.axis_index("s")
    wid = sid * NC + cid
    ROWS, DST = [rows0, rows1, rows2], [dst0, dst1, dst2]
    G, D, S = [g0, g1, g2], [d0, d1, d2], [s0, s1, s2]

    @pl.when(sid < WB_TILES)
    def _zero():
        def zrow(r, carry):
            for j in range(DOUT // 16):
                rows0[r, pl.ds(j * 16, 16)] = jnp.zeros((16,), jnp.float32)
            return carry
        lax.fori_loop(0, ZROWS, zrow, 0)
        for k in range(ROWS_PER_WB // ZROWS):
            pltpu.sync_copy(rows0.at[pl.ds(0, ZROWS)],
                            acc.at[pl.ds(sid * ROWS_PER_WB + k * ZROWS,
                                         ZROWS)])
    plsc.subcore_barrier()

    pltpu.sync_copy(src_hbm.at[wid], src_v)

    def issue(c, b):
        pltpu.async_copy(xw_hbm.at[src_v.at[c]], ROWS[b], G[b])
        pltpu.async_copy(dst_hbm.at[pl.ds(wid * PER_W + c * CHUNK, CHUNK)],
                         DST[b], D[b])

    def wait_g(b):
        pltpu.make_async_copy(xw_hbm.at[pl.ds(0, CHUNK)], ROWS[b],
                              G[b]).wait()
        pltpu.make_async_copy(dst_hbm.at[pl.ds(0, CHUNK)], DST[b],
                              D[b]).wait()

    def issue_scatter(b):
        pltpu.async_copy(ROWS[b], acc.at[DST[b]], S[b], add=True)

    def wait_s(b):
        pltpu.make_async_copy(ROWS[b], acc.at[DST[b]], S[b]).wait()

    issue(0, 0)
    issue(1, 1)
    wait_g(0)
    issue_scatter(0)
    issue(2, 2)

    def triple(o, carry):
        c = 3 * o
        wait_g(1); issue_scatter(1); wait_s(0); issue(c + 3, 0)
        wait_g(2); issue_scatter(2); wait_s(1); issue(c + 4, 1)
        wait_g(0); issue_scatter(0); wait_s(2); issue(c + 5, 2)
        return carry
    lax.fori_loop(0, (NCHUNKS - 5) // 3, triple, 0)

    wait_g(1); issue_scatter(1); wait_s(0); issue(NCHUNKS - 2, 0)
    wait_g(2); issue_scatter(2); wait_s(1); issue(NCHUNKS - 1, 1)
    wait_g(0); issue_scatter(0); wait_s(2)
    wait_g(1); issue_scatter(1); wait_s(0)
    wait_s(1)
    plsc.subcore_barrier()

    @pl.when(sid < WB_TILES)
    def _writeback():
        r0 = sid * ROWS_PER_WB
        pltpu.sync_copy(acc.at[pl.ds(r0, ROWS_PER_WB)],
                        out_hbm.at[pl.ds(cid * N + r0, ROWS_PER_WB)])


@jax.jit
def _gcn(x, edge_index, W, b):
    src = edge_index[0].reshape(NW, NCHUNKS, CHUNK)
    dst = edge_index[1]

    edge_kernel = pl.kernel(
        _edge_body,
        out_type=jax.ShapeDtypeStruct((2 * N, DIN), jnp.float32),
        mesh=plsc.VectorSubcoreMesh(core_axis_name="c", subcore_axis_name="s"),
        scratch_types=[
            pltpu.VMEM((NCHUNKS, CHUNK), jnp.int32),
            pltpu.VMEM((CHUNK,), jnp.int32),
            pltpu.VMEM((CHUNK,), jnp.int32),
            pltpu.VMEM((CHUNK,), jnp.int32),
            pltpu.VMEM((CHUNK, DIN), jnp.float32),
            pltpu.VMEM((CHUNK, DIN), jnp.float32),
            pltpu.VMEM((CHUNK, DIN), jnp.float32),
            pltpu.VMEM_SHARED((N, DIN), jnp.float32),
        ] + [pltpu.SemaphoreType.DMA] * 9,
    )
    partials = edge_kernel(x, src, dst)

    out = pl.pallas_call(
        _fin_body,
        grid=(10,),
        in_specs=[pl.BlockSpec((N // 10, DIN), lambda i: (i, 0)),
                  pl.BlockSpec((N // 10, DIN), lambda i: (i + 10, 0)),
                  pl.BlockSpec((DIN, DOUT), lambda i: (0, 0)),
                  pl.BlockSpec((1, DOUT), lambda i: (0, 0))],
        out_specs=pl.BlockSpec((N // 10, DOUT), lambda i: (i, 0)),
        out_shape=jax.ShapeDtypeStruct((N, DOUT), jnp.float32),
    )(partials, partials, W, b.reshape(1, DOUT))
    return out


def kernel(x, edge_index, edge_attr, return_attention_weights, W, b):
    out = _gcn(x, edge_index, W, b)
    return (out, (None, None))


# speedup vs baseline: 14.6157x; 1.0765x over previous
import functools

import jax
import jax.numpy as jnp
from jax import lax
from jax.experimental import pallas as pl
from jax.experimental.pallas import tpu as pltpu
from jax.experimental.pallas import tpu_sc as plsc

N = 10000
DIN = 128
DOUT = 128
E = 320000

NC = 2
NS = 16
NW = NC * NS
PER_W = E // NW
CHUNK = 80
NCHUNKS = PER_W // CHUNK
WB_ROWS = 624
WB_LAST = N - 15 * WB_ROWS
ZROWS = 40


def _fin_body(p0_ref, p1_ref, w_ref, b_ref, o_ref):
    o_ref[...] = jnp.dot(p0_ref[...] + p1_ref[...], w_ref[...],
                         preferred_element_type=jnp.float32) + b_ref[...]


def _edge_body(x_hbm, edge_hbm, dstf_hbm, out_hbm,
               src_v, dst0, dst1, dst2, rows0, rows1, rows2, acc,
               g0, g1, g2, d0, d1, d2, s0, s1, s2):
    cid = lax.axis_index("c")
    sid = lax.axis_index("s")
    wid = sid * NC + cid
    ROWS, DST = [rows0, rows1, rows2], [dst0, dst1, dst2]
    G, D, S = [g0, g1, g2], [d0, d1, d2], [s0, s1, s2]

    def issue(c, b):
        pltpu.async_copy(x_hbm.at[src_v.at[c]], ROWS[b], G[b])
        pltpu.async_copy(
            dstf_hbm.at[pl.ds(E + wid * PER_W + c * CHUNK, CHUNK)],
            DST[b], D[b])

    def wait_g(b):
        pltpu.make_async_copy(x_hbm.at[pl.ds(0, CHUNK)], ROWS[b],
                              G[b]).wait()
        pltpu.make_async_copy(dstf_hbm.at[pl.ds(0, CHUNK)], DST[b],
                              D[b]).wait()

    def issue_scatter(b):
        pltpu.async_copy(ROWS[b], acc.at[DST[b]], S[b], add=True)

    def wait_s(b):
        pltpu.make_async_copy(ROWS[b], acc.at[DST[b]], S[b]).wait()

    pltpu.sync_copy(edge_hbm.at[0, wid], src_v)
    issue(0, 0)
    issue(1, 1)

    def zrow(r, carry):
        for j in range(DIN // 16):
            rows2[r, pl.ds(j * 16, 16)] = jnp.zeros((16,), jnp.float32)
        return carry
    lax.fori_loop(0, ZROWS, zrow, 0)
    r0 = sid * WB_ROWS
    nfull = jnp.where(sid == NS - 1, WB_LAST // ZROWS, WB_ROWS // ZROWS)

    def zcopy(k, carry):
        pltpu.sync_copy(rows2.at[pl.ds(0, ZROWS)],
                        acc.at[pl.ds(r0 + k * ZROWS, ZROWS)])
        return carry
    lax.fori_loop(0, nfull, zcopy, 0)

    @pl.when(sid < NS - 1)
    def _ztail():
        pltpu.sync_copy(rows2.at[pl.ds(0, WB_ROWS % ZROWS)],
                        acc.at[pl.ds(r0 + (WB_ROWS // ZROWS) * ZROWS,
                                     WB_ROWS % ZROWS)])
    plsc.subcore_barrier()

    wait_g(0)
    issue_scatter(0)
    issue(2, 2)

    def triple(o, carry):
        c = 3 * o
        wait_g(1); issue_scatter(1); wait_s(0); issue(c + 3, 0)
        wait_g(2); issue_scatter(2); wait_s(1); issue(c + 4, 1)
        wait_g(0); issue_scatter(0); wait_s(2); issue(c + 5, 2)
        return carry
    lax.fori_loop(0, (NCHUNKS - 5) // 3, triple, 0)

    wait_g(1); issue_scatter(1); wait_s(0); issue(NCHUNKS - 2, 0)
    wait_g(2); issue_scatter(2); wait_s(1); issue(NCHUNKS - 1, 1)
    wait_g(0); issue_scatter(0); wait_s(2)
    wait_g(1); issue_scatter(1); wait_s(0)
    wait_s(1)
    plsc.subcore_barrier()

    nwb = jnp.where(sid == NS - 1, WB_LAST, WB_ROWS)
    pltpu.sync_copy(acc.at[pl.ds(r0, nwb)],
                    out_hbm.at[pl.ds(cid * N + r0, nwb)])


@jax.jit
def _gcn(x, edge_index, W, b):
    edge4 = edge_index.reshape(2, NW, NCHUNKS, CHUNK)
    edge_flat = edge_index.reshape(2 * E)

    edge_kernel = pl.kernel(
        _edge_body,
        out_type=jax.ShapeDtypeStruct((2 * N, DIN), jnp.float32),
        mesh=plsc.VectorSubcoreMesh(core_axis_name="c", subcore_axis_name="s"),
        scratch_types=[
            pltpu.VMEM((NCHUNKS, CHUNK), jnp.int32),
            pltpu.VMEM((CHUNK,), jnp.int32),
            pltpu.VMEM((CHUNK,), jnp.int32),
            pltpu.VMEM((CHUNK,), jnp.int32),
            pltpu.VMEM((CHUNK, DIN), jnp.float32),
            pltpu.VMEM((CHUNK, DIN), jnp.float32),
            pltpu.VMEM((CHUNK, DIN), jnp.float32),
            pltpu.VMEM_SHARED((N, DIN), jnp.float32),
        ] + [pltpu.SemaphoreType.DMA] * 9,
    )
    partials = edge_kernel(x, edge4, edge_flat)

    out = pl.pallas_call(
        _fin_body,
        grid=(10,),
        in_specs=[pl.BlockSpec((N // 10, DIN), lambda i: (i, 0)),
                  pl.BlockSpec((N // 10, DIN), lambda i: (i + 10, 0)),
                  pl.BlockSpec((DIN, DOUT), lambda i: (0, 0)),
                  pl.BlockSpec((1, DOUT), lambda i: (0, 0))],
        out_specs=pl.BlockSpec((N // 10, DOUT), lambda i: (i, 0)),
        out_shape=jax.ShapeDtypeStruct((N, DOUT), jnp.float32),
    )(partials, partials, W, b.reshape(1, DOUT))
    return out


def kernel(x, edge_index, edge_attr, return_attention_weights, W, b):
    out = _gcn(x, edge_index, W, b)
    return (out, (None, None))
